# baseline (speedup 1.0000x reference)
import jax
import jax.numpy as jnp
import numpy as np
from jax import lax
from jax.experimental import pallas as pl
from jax.experimental.pallas import tpu as pltpu
from jax.experimental.pallas import tpu_sc as plsc

_B = 16
_NBEAM = 4
_V = 100000
_HALF = _V // 2
_L = 16
_CH = 25
_NCH = _HALF // (_CH * _L)
_GSZ = 25
_NG = _NCH // _GSZ
_K = 8
_BIG_I = 1 << 30

_DNUMS = lax.GatherDimensionNumbers(
    offset_dims=(), collapsed_slice_dims=(0,), start_index_map=(0,))


def _bred(v, op, lane):
    for k in (1, 2, 4, 8):
        idx = (lane ^ k)[:, None]
        shuf = lax.gather(v, idx, dimension_numbers=_DNUMS, slice_sizes=(1,),
                          mode=lax.GatherScatterMode.PROMISE_IN_BOUNDS)
        v = op(v, shuf)
    return v


def _ln(x):
    b = lax.bitcast_convert_type(x, jnp.int32)
    e = ((b >> 23) & 0xFF) - 127
    m = lax.bitcast_convert_type((b & 0x7FFFFF) | 0x3F800000, jnp.float32)
    big = m >= 1.4142135
    m = jnp.where(big, m * 0.5, m)
    e = e + jnp.where(big, 1, 0)
    z = (m - 1.0) / (m + 1.0)
    z2 = z * z
    p = 2.0 * z * (1.0 + z2 * (1.0 / 3.0 + z2 *
                               (0.2 + z2 * (1.0 / 7.0 + z2 * (1.0 / 9.0)))))
    return e.astype(jnp.float32) * 0.6931471805599453 + p


def _body(flat_ref, beams_ref, outs_ref, outt_ref, outb_ref,
          bufa, bufb, sm, ssm, svsh, sish, tmpf, tmpi, mv, mi, bv,
          obs, obt, obb, sem0, sem1):
    c = lax.axis_index("c")
    s = lax.axis_index("s")
    lane = lax.iota(jnp.int32, _L)
    minf = jnp.float32(-jnp.inf)
    sems = (sem0, sem1)

    def start_copy(t):
        row_local = 2 * s + (t // 2)
        base = (32 * c + row_local) * _V + (t % 2) * _HALF
        return pltpu.async_copy(flat_ref.at[pl.ds(base, _HALF)],
                                (bufa, bufb)[t % 2], sems[t % 2])

    sub = _HALF // 5
    row0_base = (32 * c + 2 * s) * _V
    handles0 = [pltpu.async_copy(
        flat_ref.at[pl.ds(row0_base + k * sub, sub)],
        bufa.at[pl.ds(k * sub, sub)], sem0) for k in range(5)]
    handles = [None, None]
    for t in range(4):
        if t < 3:
            handles[(t + 1) % 2] = start_copy(t + 1)
        data = (bufa, bufb)[t % 2]
        half_off = (t % 2) * _HALF

        def segbody(g, _):
            base = g * (_CH * _L)
            a = [jnp.full((_L,), minf, jnp.float32) for _ in range(4)]
            for j in range(_CH):
                a[j % 4] = jnp.maximum(a[j % 4],
                                       data[pl.ds(base + j * _L, _L)])
            acc = jnp.maximum(jnp.maximum(a[0], a[1]),
                              jnp.maximum(a[2], a[3]))
            sm[pl.ds(g * _L, _L)] = acc
            return 0
        if t == 0:
            for k in range(5):
                handles0[k].wait()
                plsc.parallel_loop(25 * k, 25 * (k + 1),
                                   carry=jnp.int32(0))(segbody)
        else:
            handles[t % 2].wait()
            plsc.parallel_loop(0, _NCH, unroll=2, carry=jnp.int32(0))(segbody)

        def ssmbody(gg, _):
            acc = jnp.full((_L,), minf, jnp.float32)
            for j in range(_GSZ):
                acc = jnp.maximum(acc, sm[pl.ds((gg * _GSZ + j) * _L, _L)])
            ssm[pl.ds(gg * _L, _L)] = acc
            return 0
        lax.fori_loop(0, _NG, ssmbody, 0)

        tv = ssm[pl.ds(0, _L)]
        for gg in range(1, _NG):
            tv = jnp.maximum(tv, ssm[pl.ds(gg * _L, _L)])
        m_half = _bred(tv, jnp.maximum, lane)

        def expbody(g, accs):
            base = g * (_CH * _L)
            a = list(accs)
            for j in range(_CH):
                a[j % 4] = a[j % 4] + jnp.exp(
                    data[pl.ds(base + j * _L, _L)] - m_half)
            return tuple(a)
        z4 = tuple(jnp.zeros((_L,), jnp.float32) for _ in range(4))
        a0, a1, a2, a3 = plsc.parallel_loop(0, _NCH, unroll=2, carry=z4)(expbody)
        acc = (a0 + a1) + (a2 + a3)
        s_half = _bred(acc, jnp.add, lane)

        def extbody(i, carry):
            rv, ri = carry
            tv = ssm[pl.ds(0, _L)]
            for gg in range(1, _NG):
                tv = jnp.maximum(tv, ssm[pl.ds(gg * _L, _L)])
            mxv = _bred(tv, jnp.maximum, lane)

            ggb = jnp.full((_L,), 999, jnp.int32)
            for gg in range(_NG):
                hit = ssm[pl.ds(gg * _L, _L)] == mxv
                ggb = jnp.minimum(ggb, jnp.where(hit, gg, 999))
            gg_sel = _bred(ggb, jnp.minimum, lane)[0]

            jb = jnp.full((_L,), 999, jnp.int32)
            gbase = gg_sel * _GSZ
            for j in range(_GSZ):
                row = sm[pl.ds((gbase + j) * _L, _L)]
                jb = jnp.minimum(jb, jnp.where(row == mxv, j, 999))
            j_sel = _bred(jb, jnp.minimum, lane)[0]
            g = gg_sel * _GSZ + j_sel

            row_g = sm[pl.ds(g * _L, _L)]
            lv = _bred(jnp.where(row_g == mxv, lane, 999), jnp.minimum, lane)

            jb2 = jnp.full((_L,), 999, jnp.int32)
            dbase = g * _CH
            for j in range(_CH):
                v = data[pl.ds((dbase + j) * _L, _L)]
                hit = (v == mxv) & (lane == lv)
                jb2 = jnp.minimum(jb2, jnp.where(hit, j, 999))
            jp_sel = _bred(jb2, jnp.minimum, lane)[0]

            pos = (g * _CH + jp_sel) * _L
            vv = data[pl.ds(pos, _L)]
            data[pl.ds(pos, _L)] = jnp.where(lane == lv, minf, vv)

            ar = [jnp.full((_L,), minf, jnp.float32) for _ in range(4)]
            for j in range(_CH):
                ar[j % 4] = jnp.maximum(
                    ar[j % 4], data[pl.ds((dbase + j) * _L, _L)])
            sm[pl.ds(g * _L, _L)] = jnp.maximum(
                jnp.maximum(ar[0], ar[1]), jnp.maximum(ar[2], ar[3]))
            ag = [jnp.full((_L,), minf, jnp.float32) for _ in range(4)]
            for j in range(_GSZ):
                ag[j % 4] = jnp.maximum(
                    ag[j % 4], sm[pl.ds((gbase + j) * _L, _L)])
            ssm[pl.ds(gg_sel * _L, _L)] = jnp.maximum(
                jnp.maximum(ag[0], ag[1]), jnp.maximum(ag[2], ag[3]))

            elemv = g * (_CH * _L) + jp_sel * _L + lv
            rv = jnp.where(lane == i, mxv, rv)
            ri = jnp.where(lane == i, elemv + half_off, ri)
            return rv, ri

        rv0 = jnp.full((_L,), minf, jnp.float32)
        ri0 = jnp.zeros((_L,), jnp.int32)
        rv, ri = lax.fori_loop(0, _K, extbody, (rv0, ri0))

        rv = jnp.where(lane == 8, m_half, rv)
        rv = jnp.where(lane == 9, s_half, rv)
        tmpf[...] = rv
        tmpi[...] = ri
        hr = 4 * s + t
        pltpu.sync_copy(tmpf, svsh.at[pl.ds(hr * _L, _L)])
        pltpu.sync_copy(tmpi, sish.at[pl.ds(hr * _L, _L)])

    plsc.subcore_barrier()

    s_m = s % 8
    b = 8 * c + s_m
    pltpu.sync_copy(svsh.at[pl.ds(8 * s_m * _L, 8 * _L)], mv)
    pltpu.sync_copy(sish.at[pl.ds(8 * s_m * _L, 8 * _L)], mi)
    pltpu.sync_copy(beams_ref, bv.at[pl.ds(0, 64)])

    zs = jnp.zeros((_L,), jnp.float32)
    mh0, mh1, sh0, sh1 = zs, zs, zs, zs
    for k in range(_NBEAM):
        vh0 = mv[pl.ds(2 * k * _L, _L)]
        vh1 = mv[pl.ds((2 * k + 1) * _L, _L)]
        mh0 = jnp.where(lane == k, vh0[8], mh0)
        mh1 = jnp.where(lane == k, vh1[8], mh1)
        sh0 = jnp.where(lane == k, vh0[9], sh0)
        sh1 = jnp.where(lane == k, vh1[9], sh1)
    mk = jnp.maximum(mh0, mh1)
    svec = sh0 * jnp.exp(mh0 - mk) + sh1 * jnp.exp(mh1 - mk)
    lns = _ln(svec)
    vb = bv[pl.ds(4 * b, _L)]
    adj = vb - mk - lns

    svs, fls = [], []
    for h in range(8):
        k = h // 2
        vals = mv[pl.ds(h * _L, _L)]
        idxs = mi[pl.ds(h * _L, _L)]
        a_k = _bred(jnp.where(lane == k, adj, minf), jnp.maximum, lane)
        sc = jnp.where(lane < _K, vals + a_k, minf)
        fl = jnp.where(lane < _K, idxs + k * _V, _BIG_I)
        svs.append(sc)
        fls.append(fl)

    rs = jnp.full((_L,), minf, jnp.float32)
    rf = jnp.zeros((_L,), jnp.int32)
    for i in range(_K):
        m = svs[0]
        for h in range(1, 8):
            m = jnp.maximum(m, svs[h])
        gm = _bred(m, jnp.maximum, lane)
        fm = jnp.full((_L,), _BIG_I, jnp.int32)
        for h in range(8):
            fm = jnp.minimum(fm, jnp.where(svs[h] == gm, fls[h], _BIG_I))
        gmf = _bred(fm, jnp.minimum, lane)
        for h in range(8):
            svs[h] = jnp.where((svs[h] == gm) & (fls[h] == gmf),
                               minf, svs[h])
        rs = jnp.where(lane == i, gm, rs)
        rf = jnp.where(lane == i, gmf, rf)

    obs[...] = rs
    bm = (jnp.where(rf >= _V, 1, 0) + jnp.where(rf >= 2 * _V, 1, 0)
          + jnp.where(rf >= 3 * _V, 1, 0))
    obt[...] = rf - bm * _V
    obb[...] = bm

    @pl.when(s < 8)
    def _write_out():
        pltpu.sync_copy(obs, outs_ref.at[pl.ds(b * _L, _L)])
        pltpu.sync_copy(obt, outt_ref.at[pl.ds(b * _L, _L)])
        pltpu.sync_copy(obb, outb_ref.at[pl.ds(b * _L, _L)])


_scall = pl.kernel(
    _body,
    out_type=(
        jax.ShapeDtypeStruct((_B * _L,), jnp.float32),
        jax.ShapeDtypeStruct((_B * _L,), jnp.int32),
        jax.ShapeDtypeStruct((_B * _L,), jnp.int32),
    ),
    mesh=plsc.VectorSubcoreMesh(core_axis_name="c", subcore_axis_name="s"),
    scratch_types=[
        pltpu.VMEM((_HALF,), jnp.float32),
        pltpu.VMEM((_HALF,), jnp.float32),
        pltpu.VMEM((_NCH * _L,), jnp.float32),
        pltpu.VMEM((_NG * _L,), jnp.float32),
        pltpu.VMEM_SHARED((64 * _L,), jnp.float32),
        pltpu.VMEM_SHARED((64 * _L,), jnp.int32),
        pltpu.VMEM((_L,), jnp.float32),
        pltpu.VMEM((_L,), jnp.int32),
        pltpu.VMEM((8 * _L,), jnp.float32),
        pltpu.VMEM((8 * _L,), jnp.int32),
        pltpu.VMEM((80,), jnp.float32),
        pltpu.VMEM((_L,), jnp.float32),
        pltpu.VMEM((_L,), jnp.int32),
        pltpu.VMEM((_L,), jnp.int32),
        pltpu.SemaphoreType.DMA,
        pltpu.SemaphoreType.DMA,
    ],
)


def kernel(logits, beam_scores):
    flat = logits.reshape(-1)
    outs, outt, outb = _scall(flat, beam_scores)
    outs = outs.reshape(_B, _L)[:, :_K]
    outt = outt.reshape(_B, _L)[:, :_K]
    outb = outb.reshape(_B, _L)[:, :_K]
    return outs, outt, outb


# speedup vs baseline: 1.0041x; 1.0041x over previous
import jax
import jax.numpy as jnp
from jax import lax
from jax.experimental import pallas as pl
from jax.experimental.pallas import tpu as pltpu
from jax.experimental.pallas import tpu_sc as plsc

_B = 16
_NBEAM = 4
_V = 100000
_HALF = _V // 2
_L = 16
_CH = 25
_NCH = _HALF // (_CH * _L)
_GSZ = 25
_NG = _NCH // _GSZ
_K = 8
_BIG_I = 1 << 30

_DNUMS = lax.GatherDimensionNumbers(
    offset_dims=(), collapsed_slice_dims=(0,), start_index_map=(0,))


def _bred(v, op, lane):
    for k in (1, 2, 4, 8):
        idx = (lane ^ k)[:, None]
        shuf = lax.gather(v, idx, dimension_numbers=_DNUMS, slice_sizes=(1,),
                          mode=lax.GatherScatterMode.PROMISE_IN_BOUNDS)
        v = op(v, shuf)
    return v


def _ln(x):
    b = lax.bitcast_convert_type(x, jnp.int32)
    e = ((b >> 23) & 0xFF) - 127
    m = lax.bitcast_convert_type((b & 0x7FFFFF) | 0x3F800000, jnp.float32)
    big = m >= 1.4142135
    m = jnp.where(big, m * 0.5, m)
    e = e + jnp.where(big, 1, 0)
    z = (m - 1.0) / (m + 1.0)
    z2 = z * z
    p = 2.0 * z * (1.0 + z2 * (1.0 / 3.0 + z2 *
                               (0.2 + z2 * (1.0 / 7.0 + z2 * (1.0 / 9.0)))))
    return e.astype(jnp.float32) * 0.6931471805599453 + p


def _body(flat_ref, beams_ref, outs_ref, outt_ref, outb_ref,
          bufa, bufb, sm, ssm, svsh, sish, tmpf, tmpi, mv, mi, bv,
          obs, obt, obb, sem0, sem1):
    c = lax.axis_index("c")
    s = lax.axis_index("s")
    lane = lax.iota(jnp.int32, _L)
    minf = jnp.float32(-jnp.inf)
    sems = (sem0, sem1)

    def start_copy(t):
        row_local = 2 * s + (t // 2)
        base = (32 * c + row_local) * _V + (t % 2) * _HALF
        return pltpu.async_copy(flat_ref.at[pl.ds(base, _HALF)],
                                (bufa, bufb)[t % 2], sems[t % 2])

    sub = _HALF // 5
    row0_base = (32 * c + 2 * s) * _V
    handles0 = [pltpu.async_copy(
        flat_ref.at[pl.ds(row0_base + k * sub, sub)],
        bufa.at[pl.ds(k * sub, sub)], sem0) for k in range(5)]
    handles = [None, None]
    for t in range(4):
        if t < 3:
            handles[(t + 1) % 2] = start_copy(t + 1)
        data = (bufa, bufb)[t % 2]
        half_off = (t % 2) * _HALF

        def segbody(g, _):
            base = g * (_CH * _L)
            a = [jnp.full((_L,), minf, jnp.float32) for _ in range(4)]
            for j in range(_CH):
                a[j % 4] = jnp.maximum(a[j % 4],
                                       data[pl.ds(base + j * _L, _L)])
            acc = jnp.maximum(jnp.maximum(a[0], a[1]),
                              jnp.maximum(a[2], a[3]))
            sm[pl.ds(g * _L, _L)] = acc
            return 0
        if t == 0:
            for k in range(5):
                handles0[k].wait()
                plsc.parallel_loop(25 * k, 25 * (k + 1),
                                   carry=jnp.int32(0))(segbody)
        else:
            handles[t % 2].wait()
            plsc.parallel_loop(0, _NCH, carry=jnp.int32(0))(segbody)

        def ssmbody(gg, _):
            acc = jnp.full((_L,), minf, jnp.float32)
            for j in range(_GSZ):
                acc = jnp.maximum(acc, sm[pl.ds((gg * _GSZ + j) * _L, _L)])
            ssm[pl.ds(gg * _L, _L)] = acc
            return 0
        lax.fori_loop(0, _NG, ssmbody, 0)

        tv = ssm[pl.ds(0, _L)]
        for gg in range(1, _NG):
            tv = jnp.maximum(tv, ssm[pl.ds(gg * _L, _L)])
        m_half = _bred(tv, jnp.maximum, lane)

        def expbody(g, accs):
            base = g * (_CH * _L)
            a = list(accs)
            for j in range(_CH):
                a[j % 4] = a[j % 4] + jnp.exp(
                    data[pl.ds(base + j * _L, _L)] - m_half)
            return tuple(a)
        z4 = tuple(jnp.zeros((_L,), jnp.float32) for _ in range(4))
        a0, a1, a2, a3 = plsc.parallel_loop(0, _NCH, carry=z4)(expbody)
        acc = (a0 + a1) + (a2 + a3)
        s_half = _bred(acc, jnp.add, lane)

        def extbody(i, carry):
            rv, ri = carry
            tv = ssm[pl.ds(0, _L)]
            for gg in range(1, _NG):
                tv = jnp.maximum(tv, ssm[pl.ds(gg * _L, _L)])
            mxv = _bred(tv, jnp.maximum, lane)

            ggb = jnp.full((_L,), 999, jnp.int32)
            for gg in range(_NG):
                hit = ssm[pl.ds(gg * _L, _L)] == mxv
                ggb = jnp.minimum(ggb, jnp.where(hit, gg, 999))
            gg_sel = _bred(ggb, jnp.minimum, lane)[0]

            jb = jnp.full((_L,), 999, jnp.int32)
            gbase = gg_sel * _GSZ
            for j in range(_GSZ):
                row = sm[pl.ds((gbase + j) * _L, _L)]
                jb = jnp.minimum(jb, jnp.where(row == mxv, j, 999))
            j_sel = _bred(jb, jnp.minimum, lane)[0]
            g = gg_sel * _GSZ + j_sel

            row_g = sm[pl.ds(g * _L, _L)]
            lv = _bred(jnp.where(row_g == mxv, lane, 999), jnp.minimum, lane)

            jb2 = jnp.full((_L,), 999, jnp.int32)
            dbase = g * _CH
            for j in range(_CH):
                v = data[pl.ds((dbase + j) * _L, _L)]
                hit = (v == mxv) & (lane == lv)
                jb2 = jnp.minimum(jb2, jnp.where(hit, j, 999))
            jp_sel = _bred(jb2, jnp.minimum, lane)[0]

            pos = (g * _CH + jp_sel) * _L
            vv = data[pl.ds(pos, _L)]
            data[pl.ds(pos, _L)] = jnp.where(lane == lv, minf, vv)

            ar = [jnp.full((_L,), minf, jnp.float32) for _ in range(4)]
            for j in range(_CH):
                ar[j % 4] = jnp.maximum(
                    ar[j % 4], data[pl.ds((dbase + j) * _L, _L)])
            sm[pl.ds(g * _L, _L)] = jnp.maximum(
                jnp.maximum(ar[0], ar[1]), jnp.maximum(ar[2], ar[3]))
            ag = [jnp.full((_L,), minf, jnp.float32) for _ in range(4)]
            for j in range(_GSZ):
                ag[j % 4] = jnp.maximum(
                    ag[j % 4], sm[pl.ds((gbase + j) * _L, _L)])
            ssm[pl.ds(gg_sel * _L, _L)] = jnp.maximum(
                jnp.maximum(ag[0], ag[1]), jnp.maximum(ag[2], ag[3]))

            elemv = g * (_CH * _L) + jp_sel * _L + lv
            rv = jnp.where(lane == i, mxv, rv)
            ri = jnp.where(lane == i, elemv + half_off, ri)
            return rv, ri

        rv0 = jnp.full((_L,), minf, jnp.float32)
        ri0 = jnp.zeros((_L,), jnp.int32)
        rv, ri = lax.fori_loop(0, _K, extbody, (rv0, ri0))

        rv = jnp.where(lane == 8, m_half, rv)
        rv = jnp.where(lane == 9, s_half, rv)
        tmpf[...] = rv
        tmpi[...] = ri
        hr = 4 * s + t
        pltpu.sync_copy(tmpf, svsh.at[pl.ds(hr * _L, _L)])
        pltpu.sync_copy(tmpi, sish.at[pl.ds(hr * _L, _L)])

    plsc.subcore_barrier()

    s_m = s % 8
    b = 8 * c + s_m
    pltpu.sync_copy(svsh.at[pl.ds(8 * s_m * _L, 8 * _L)], mv)
    pltpu.sync_copy(sish.at[pl.ds(8 * s_m * _L, 8 * _L)], mi)
    pltpu.sync_copy(beams_ref, bv.at[pl.ds(0, 64)])

    zs = jnp.zeros((_L,), jnp.float32)
    mh0, mh1, sh0, sh1 = zs, zs, zs, zs
    for k in range(_NBEAM):
        vh0 = mv[pl.ds(2 * k * _L, _L)]
        vh1 = mv[pl.ds((2 * k + 1) * _L, _L)]
        mh0 = jnp.where(lane == k, vh0[8], mh0)
        mh1 = jnp.where(lane == k, vh1[8], mh1)
        sh0 = jnp.where(lane == k, vh0[9], sh0)
        sh1 = jnp.where(lane == k, vh1[9], sh1)
    mk = jnp.maximum(mh0, mh1)
    svec = sh0 * jnp.exp(mh0 - mk) + sh1 * jnp.exp(mh1 - mk)
    lns = _ln(svec)
    vb = bv[pl.ds(4 * b, _L)]
    adj = vb - mk - lns

    svs, fls = [], []
    for h in range(8):
        k = h // 2
        vals = mv[pl.ds(h * _L, _L)]
        idxs = mi[pl.ds(h * _L, _L)]
        a_k = _bred(jnp.where(lane == k, adj, minf), jnp.maximum, lane)
        sc = jnp.where(lane < _K, vals + a_k, minf)
        fl = jnp.where(lane < _K, idxs + k * _V, _BIG_I)
        svs.append(sc)
        fls.append(fl)

    rs = jnp.full((_L,), minf, jnp.float32)
    rf = jnp.zeros((_L,), jnp.int32)
    for i in range(_K):
        m = svs[0]
        for h in range(1, 8):
            m = jnp.maximum(m, svs[h])
        gm = _bred(m, jnp.maximum, lane)
        fm = jnp.full((_L,), _BIG_I, jnp.int32)
        for h in range(8):
            fm = jnp.minimum(fm, jnp.where(svs[h] == gm, fls[h], _BIG_I))
        gmf = _bred(fm, jnp.minimum, lane)
        for h in range(8):
            svs[h] = jnp.where((svs[h] == gm) & (fls[h] == gmf),
                               minf, svs[h])
        rs = jnp.where(lane == i, gm, rs)
        rf = jnp.where(lane == i, gmf, rf)

    obs[...] = rs
    bm = (jnp.where(rf >= _V, 1, 0) + jnp.where(rf >= 2 * _V, 1, 0)
          + jnp.where(rf >= 3 * _V, 1, 0))
    obt[...] = rf - bm * _V
    obb[...] = bm

    @pl.when(s < 8)
    def _write_out():
        pltpu.sync_copy(obs, outs_ref.at[pl.ds(b * _L, _L)])
        pltpu.sync_copy(obt, outt_ref.at[pl.ds(b * _L, _L)])
        pltpu.sync_copy(obb, outb_ref.at[pl.ds(b * _L, _L)])


_scall = pl.kernel(
    _body,
    out_type=(
        jax.ShapeDtypeStruct((_B * _L,), jnp.float32),
        jax.ShapeDtypeStruct((_B * _L,), jnp.int32),
        jax.ShapeDtypeStruct((_B * _L,), jnp.int32),
    ),
    mesh=plsc.VectorSubcoreMesh(core_axis_name="c", subcore_axis_name="s"),
    scratch_types=[
        pltpu.VMEM((_HALF,), jnp.float32),
        pltpu.VMEM((_HALF,), jnp.float32),
        pltpu.VMEM((_NCH * _L,), jnp.float32),
        pltpu.VMEM((_NG * _L,), jnp.float32),
        pltpu.VMEM_SHARED((64 * _L,), jnp.float32),
        pltpu.VMEM_SHARED((64 * _L,), jnp.int32),
        pltpu.VMEM((_L,), jnp.float32),
        pltpu.VMEM((_L,), jnp.int32),
        pltpu.VMEM((8 * _L,), jnp.float32),
        pltpu.VMEM((8 * _L,), jnp.int32),
        pltpu.VMEM((80,), jnp.float32),
        pltpu.VMEM((_L,), jnp.float32),
        pltpu.VMEM((_L,), jnp.int32),
        pltpu.VMEM((_L,), jnp.int32),
        pltpu.SemaphoreType.DMA,
        pltpu.SemaphoreType.DMA,
    ],
)


def kernel(logits, beam_scores):
    flat = logits.reshape(-1)
    outs, outt, outb = _scall(flat, beam_scores)
    outs = outs.reshape(_B, _L)[:, :_K]
    outt = outt.reshape(_B, _L)[:, :_K]
    outb = outb.reshape(_B, _L)[:, :_K]
    return outs, outt, outb
